# baseline (speedup 1.0000x reference)
import jax
import jax.numpy as jnp
from jax.experimental import pallas as pl


_BLOCK_N = 1000


def _sumdis_kernel(dist_ref, feat_ref, out_ref):
    d = dist_ref[...]
    norm = jnp.sum(d, axis=1, keepdims=True)
    w = d / norm
    f = feat_ref[...]
    z = (w[:, 0:1] * f[:, 0, :]
         + w[:, 1:2] * f[:, 1, :]
         + w[:, 2:3] * f[:, 2, :])
    out_ref[...] = z


def kernel(distance, interpolated_feature):
    N, K = distance.shape
    D = interpolated_feature.shape[-1]
    B = _BLOCK_N
    grid = (N // B,)
    return pl.pallas_call(
        _sumdis_kernel,
        grid=grid,
        in_specs=[
            pl.BlockSpec((B, K), lambda i: (i, 0)),
            pl.BlockSpec((B, K, D), lambda i: (i, 0, 0)),
        ],
        out_specs=pl.BlockSpec((B, D), lambda i: (i, 0)),
        out_shape=jax.ShapeDtypeStruct((N, D), interpolated_feature.dtype),
    )(distance, interpolated_feature)


# speedup vs baseline: 1.2779x; 1.2779x over previous
import jax
import jax.numpy as jnp
from jax.experimental import pallas as pl


_BLOCK_N = 2000


def _sumdis_kernel(dist_ref, feat_ref, out_ref):
    d = dist_ref[...]
    norm = jnp.sum(d, axis=1, keepdims=True)
    w = d / norm
    f = feat_ref[...]
    D = out_ref.shape[-1]
    z = (w[:, 0:1] * f[:, 0:D]
         + w[:, 1:2] * f[:, D:2 * D]
         + w[:, 2:3] * f[:, 2 * D:3 * D])
    out_ref[...] = z


def kernel(distance, interpolated_feature):
    N, K = distance.shape
    D = interpolated_feature.shape[-1]
    feat2 = interpolated_feature.reshape(N, K * D)
    B = _BLOCK_N
    grid = (N // B,)
    return pl.pallas_call(
        _sumdis_kernel,
        grid=grid,
        in_specs=[
            pl.BlockSpec((B, K), lambda i: (i, 0)),
            pl.BlockSpec((B, K * D), lambda i: (i, 0)),
        ],
        out_specs=pl.BlockSpec((B, D), lambda i: (i, 0)),
        out_shape=jax.ShapeDtypeStruct((N, D), interpolated_feature.dtype),
    )(distance, feat2)
